# Initial kernel scaffold; baseline (speedup 1.0000x reference)
import functools

import jax
import jax.numpy as jnp
from jax import lax
from jax.experimental import pallas as pl
from jax.experimental.pallas import tpu as pltpu
from jax.experimental.pallas import tpu_sc as plsc

N, E, D, FD = 10000, 320000, 128, 64
NC, NS = 2, 16
NW = NC * NS
EPW = E // NW
C = 80
NPT = N // NS
CW = 16


def _gather_body(x_hbm, tgt_hbm, src_hbm, tf_hbm, sf_hbm,
                 idx_t, idx_s, rows_t, rows_s, sem):
    wid = lax.axis_index("s") * NC + lax.axis_index("c")
    base = wid * EPW

    def body(i, carry):
        off = base + i * C
        pltpu.sync_copy(tgt_hbm.at[pl.ds(off, C)], idx_t)
        pltpu.sync_copy(src_hbm.at[pl.ds(off, C)], idx_s)
        cp_t = pltpu.async_copy(x_hbm.at[idx_t], rows_t, sem)
        cp_t.wait()
        cp_s = pltpu.async_copy(x_hbm.at[idx_s], rows_s, sem)
        cp_s.wait()
        pltpu.sync_copy(rows_t, tf_hbm.at[pl.ds(off, C)])
        pltpu.sync_copy(rows_s, sf_hbm.at[pl.ds(off, C)])
        return carry

    lax.fori_loop(0, EPW // C, body, 0)


def _scatter_body(msg_hbm, tgt_hbm, zero_agg_hbm, zero_cnt_hbm, ones_hbm,
                  agg_hbm, cnt_hbm,
                  idx_v, rows_v, ones_v, wb_v, wbc_v, agg_sh, cnt_sh, sem):
    cid = lax.axis_index("c")
    sid = lax.axis_index("s")
    wid = sid * NC + cid
    pltpu.sync_copy(zero_agg_hbm.at[pl.ds(sid * NPT, NPT)],
                    agg_sh.at[pl.ds(sid * NPT, NPT)])
    pltpu.sync_copy(zero_cnt_hbm.at[pl.ds(sid * NPT, NPT)],
                    cnt_sh.at[pl.ds(sid * NPT, NPT)])
    pltpu.sync_copy(ones_hbm, ones_v)
    plsc.subcore_barrier()

    base = wid * EPW

    def body(i, carry):
        off = base + i * C
        pltpu.sync_copy(tgt_hbm.at[pl.ds(off, C)], idx_v)
        pltpu.sync_copy(msg_hbm.at[pl.ds(off, C)], rows_v)
        pltpu.sync_copy(rows_v, agg_sh.at[idx_v], add=True)
        pltpu.sync_copy(ones_v, cnt_sh.at[idx_v], add=True)
        return carry

    lax.fori_loop(0, EPW // C, body, 0)
    plsc.subcore_barrier()
    pltpu.sync_copy(agg_sh.at[pl.ds(sid * NPT, NPT)], wb_v)
    pltpu.sync_copy(wb_v, agg_hbm.at[cid, pl.ds(sid * NPT, NPT)])
    pltpu.sync_copy(cnt_sh.at[pl.ds(sid * NPT, NPT)], wbc_v)
    pltpu.sync_copy(wbc_v, cnt_hbm.at[cid, pl.ds(sid * NPT, NPT)])


def _dense_body(tf_ref, sf_ref, attn_ref, g1_ref, g2_ref, b1_ref, b2_ref,
                w1_ref, w2_ref, bias_ref, out_ref):
    tf = tf_ref[...]
    sf = sf_ref[...]
    s = jnp.sum(tf, axis=1, keepdims=True) + jnp.sum(sf, axis=1, keepdims=True)
    sq = (jnp.sum(tf * tf, axis=1, keepdims=True)
          + jnp.sum(sf * sf, axis=1, keepdims=True))
    mu = s * (1.0 / (2 * D))
    var = sq * (1.0 / (2 * D)) - mu * mu
    inv = lax.rsqrt(var + 1e-5)
    ht = jnp.maximum((tf - mu) * inv * g1_ref[...] + b1_ref[...], 0.0)
    hs = jnp.maximum((sf - mu) * inv * g2_ref[...] + b2_ref[...], 0.0)
    z = (jnp.dot(ht, w1_ref[...], preferred_element_type=jnp.float32)
         + jnp.dot(hs, w2_ref[...], preferred_element_type=jnp.float32)
         + bias_ref[...])
    gate = jnp.mean(jax.nn.sigmoid(z), axis=1, keepdims=True)
    out_ref[...] = sf * (gate * attn_ref[...])


def _combine_body(agg_ref, cnt_ref, out_ref):
    a = agg_ref[0] + agg_ref[1]
    c = cnt_ref[0, :, 0:1] + cnt_ref[1, :, 0:1]
    out_ref[...] = a / jnp.maximum(c, 1.0)


def kernel(x, edge_index, attn_value, ln_gamma, ln_beta, W, b):
    ei = edge_index.astype(jnp.int32)
    tgt = ei[0]
    src = ei[1]

    mesh = plsc.VectorSubcoreMesh(core_axis_name="c", subcore_axis_name="s")

    gather = pl.kernel(
        _gather_body,
        out_type=[jax.ShapeDtypeStruct((E, D), jnp.float32),
                  jax.ShapeDtypeStruct((E, D), jnp.float32)],
        scratch_types=[pltpu.VMEM((C,), jnp.int32),
                       pltpu.VMEM((C,), jnp.int32),
                       pltpu.VMEM((C, D), jnp.float32),
                       pltpu.VMEM((C, D), jnp.float32),
                       pltpu.SemaphoreType.DMA],
        mesh=mesh,
    )
    tf, sf = gather(x, tgt, src)

    B = 2000
    g1 = ln_gamma[:D].reshape(1, D)
    g2 = ln_gamma[D:].reshape(1, D)
    b1 = ln_beta[:D].reshape(1, D)
    b2 = ln_beta[D:].reshape(1, D)
    w1 = W[:, :D].T
    w2 = W[:, D:].T
    bias = b.reshape(1, FD)
    attn2d = attn_value.reshape(E, 1)

    row_spec = pl.BlockSpec((B, D), lambda i: (i, 0))
    full = lambda shape: pl.BlockSpec(shape, lambda i: tuple(0 for _ in shape))
    msg = pl.pallas_call(
        _dense_body,
        grid=(E // B,),
        in_specs=[row_spec, row_spec,
                  pl.BlockSpec((B, 1), lambda i: (i, 0)),
                  full((1, D)), full((1, D)), full((1, D)), full((1, D)),
                  full((D, FD)), full((D, FD)), full((1, FD))],
        out_specs=row_spec,
        out_shape=jax.ShapeDtypeStruct((E, D), jnp.float32),
    )(tf, sf, attn2d, g1, g2, b1, b2, w1, w2, bias)

    zero_agg = jnp.zeros((N, D), jnp.float32)
    zero_cnt = jnp.zeros((N, CW), jnp.float32)
    ones_rows = jnp.ones((C, CW), jnp.float32)
    scatter = pl.kernel(
        _scatter_body,
        out_type=[jax.ShapeDtypeStruct((NC, N, D), jnp.float32),
                  jax.ShapeDtypeStruct((NC, N, CW), jnp.float32)],
        scratch_types=[pltpu.VMEM((C,), jnp.int32),
                       pltpu.VMEM((C, D), jnp.float32),
                       pltpu.VMEM((C, CW), jnp.float32),
                       pltpu.VMEM((NPT, D), jnp.float32),
                       pltpu.VMEM((NPT, CW), jnp.float32),
                       pltpu.VMEM_SHARED((N, D), jnp.float32),
                       pltpu.VMEM_SHARED((N, CW), jnp.float32),
                       pltpu.SemaphoreType.DMA],
        mesh=mesh,
    )
    agg, cnt = scatter(msg, tgt, zero_agg, zero_cnt, ones_rows)

    Bn = 2000
    out = pl.pallas_call(
        _combine_body,
        grid=(N // Bn,),
        in_specs=[pl.BlockSpec((NC, Bn, D), lambda i: (0, i, 0)),
                  pl.BlockSpec((NC, Bn, CW), lambda i: (0, i, 0))],
        out_specs=pl.BlockSpec((Bn, D), lambda i: (i, 0)),
        out_shape=jax.ShapeDtypeStruct((N, D), jnp.float32),
    )(agg, cnt)
    return out


# speedup vs baseline: 2.6682x; 2.6682x over previous
import functools

import jax
import jax.numpy as jnp
from jax import lax
from jax.experimental import pallas as pl
from jax.experimental.pallas import tpu as pltpu
from jax.experimental.pallas import tpu_sc as plsc

N, E, D, FD = 10000, 320000, 128, 64
NC, NS = 2, 16
NW = NC * NS
EPW = E // NW
C = 80
NP = 10240
NPT = NP // NS
CW = 16
HD = D // 2
EPT = E // NS


def _gather_body(x_hbm, tgt_hbm, src_hbm, tf_hbm, sf_hbm,
                 idx_t, idx_s, rows_t, rows_s, sem):
    wid = lax.axis_index("s") * NC + lax.axis_index("c")
    base = wid * EPW

    def body(i, carry):
        off = base + i * C
        pltpu.sync_copy(tgt_hbm.at[pl.ds(off, C)], idx_t)
        pltpu.sync_copy(src_hbm.at[pl.ds(off, C)], idx_s)
        cp_t = pltpu.async_copy(x_hbm.at[idx_t], rows_t, sem)
        cp_t.wait()
        cp_s = pltpu.async_copy(x_hbm.at[idx_s], rows_s, sem)
        cp_s.wait()
        pltpu.sync_copy(rows_t, tf_hbm.at[pl.ds(off, C)])
        pltpu.sync_copy(rows_s, sf_hbm.at[pl.ds(off, C)])
        return carry

    lax.fori_loop(0, EPW // C, body, 0)


def _scatter_body(msg_hbm, tgt_hbm, zero_agg_hbm, zero_cnt_hbm, ones_hbm,
                  agg_hbm, cnt_hbm,
                  idx_v, rows_v, ones_v, wb_v, wbc_v, agg_sh, cnt_sh, sem):
    cid = lax.axis_index("c")
    sid = lax.axis_index("s")
    pltpu.sync_copy(zero_agg_hbm, agg_sh.at[pl.ds(sid * NPT, NPT)])
    pltpu.sync_copy(zero_cnt_hbm, cnt_sh.at[pl.ds(sid * NPT, NPT)])
    pltpu.sync_copy(ones_hbm, ones_v)
    plsc.subcore_barrier()

    base = sid * EPT
    col0 = cid * HD

    def body(i, carry):
        off = base + i * C
        pltpu.sync_copy(tgt_hbm.at[pl.ds(off, C)], idx_v)
        pltpu.sync_copy(msg_hbm.at[pl.ds(off, C), pl.ds(col0, HD)], rows_v)
        pltpu.sync_copy(rows_v, agg_sh.at[idx_v], add=True)

        @pl.when(cid == 0)
        def _():
            pltpu.sync_copy(ones_v, cnt_sh.at[idx_v], add=True)
        return carry

    lax.fori_loop(0, EPT // C, body, 0)
    plsc.subcore_barrier()
    pltpu.sync_copy(agg_sh.at[pl.ds(sid * NPT, NPT)], wb_v)
    pltpu.sync_copy(wb_v, agg_hbm.at[cid, pl.ds(sid * NPT, NPT)])
    pltpu.sync_copy(cnt_sh.at[pl.ds(sid * NPT, NPT)], wbc_v)
    pltpu.sync_copy(wbc_v, cnt_hbm.at[cid, pl.ds(sid * NPT, NPT)])


def _dense_body(tf_ref, sf_ref, attn_ref, g1_ref, g2_ref, b1_ref, b2_ref,
                w1_ref, w2_ref, bias_ref, out_ref):
    tf = tf_ref[...]
    sf = sf_ref[...]
    s = jnp.sum(tf, axis=1, keepdims=True) + jnp.sum(sf, axis=1, keepdims=True)
    sq = (jnp.sum(tf * tf, axis=1, keepdims=True)
          + jnp.sum(sf * sf, axis=1, keepdims=True))
    mu = s * (1.0 / (2 * D))
    var = sq * (1.0 / (2 * D)) - mu * mu
    inv = lax.rsqrt(var + 1e-5)
    ht = jnp.maximum((tf - mu) * inv * g1_ref[...] + b1_ref[...], 0.0)
    hs = jnp.maximum((sf - mu) * inv * g2_ref[...] + b2_ref[...], 0.0)
    z = (jnp.dot(ht, w1_ref[...], preferred_element_type=jnp.float32)
         + jnp.dot(hs, w2_ref[...], preferred_element_type=jnp.float32)
         + bias_ref[...])
    gate = jnp.mean(jax.nn.sigmoid(z), axis=1, keepdims=True)
    out_ref[...] = sf * (gate * attn_ref[...])


def _combine_body(agg_ref, cnt_ref, out_ref):
    a = jnp.concatenate([agg_ref[0], agg_ref[1]], axis=1)
    c = cnt_ref[0, :, 0:1]
    out_ref[...] = a / jnp.maximum(c, 1.0)


def kernel(x, edge_index, attn_value, ln_gamma, ln_beta, W, b):
    ei = edge_index.astype(jnp.int32)
    tgt = ei[0]
    src = ei[1]

    mesh = plsc.VectorSubcoreMesh(core_axis_name="c", subcore_axis_name="s")

    gather = pl.kernel(
        _gather_body,
        out_type=[jax.ShapeDtypeStruct((E, D), jnp.float32),
                  jax.ShapeDtypeStruct((E, D), jnp.float32)],
        scratch_types=[pltpu.VMEM((C,), jnp.int32),
                       pltpu.VMEM((C,), jnp.int32),
                       pltpu.VMEM((C, D), jnp.float32),
                       pltpu.VMEM((C, D), jnp.float32),
                       pltpu.SemaphoreType.DMA],
        mesh=mesh,
        compiler_params=pltpu.CompilerParams(use_tc_tiling_on_sc=False),
    )
    tf, sf = gather(x, tgt, src)

    B = 2000
    g1 = ln_gamma[:D].reshape(1, D)
    g2 = ln_gamma[D:].reshape(1, D)
    b1 = ln_beta[:D].reshape(1, D)
    b2 = ln_beta[D:].reshape(1, D)
    w1 = W[:, :D].T
    w2 = W[:, D:].T
    bias = b.reshape(1, FD)
    attn2d = attn_value.reshape(E, 1)

    row_spec = pl.BlockSpec((B, D), lambda i: (i, 0))
    full = lambda shape: pl.BlockSpec(shape, lambda i: tuple(0 for _ in shape))
    msg = pl.pallas_call(
        _dense_body,
        grid=(E // B,),
        in_specs=[row_spec, row_spec,
                  pl.BlockSpec((B, 1), lambda i: (i, 0)),
                  full((1, D)), full((1, D)), full((1, D)), full((1, D)),
                  full((D, FD)), full((D, FD)), full((1, FD))],
        out_specs=row_spec,
        out_shape=jax.ShapeDtypeStruct((E, D), jnp.float32),
    )(tf, sf, attn2d, g1, g2, b1, b2, w1, w2, bias)

    zero_agg = jnp.zeros((NPT, HD), jnp.float32)
    zero_cnt = jnp.zeros((NPT, CW), jnp.float32)
    ones_rows = jnp.ones((C, CW), jnp.float32)
    scatter = pl.kernel(
        _scatter_body,
        out_type=[jax.ShapeDtypeStruct((NC, NP, HD), jnp.float32),
                  jax.ShapeDtypeStruct((NC, NP, CW), jnp.float32)],
        scratch_types=[pltpu.VMEM((C,), jnp.int32),
                       pltpu.VMEM((C, HD), jnp.float32),
                       pltpu.VMEM((C, CW), jnp.float32),
                       pltpu.VMEM((NPT, HD), jnp.float32),
                       pltpu.VMEM((NPT, CW), jnp.float32),
                       pltpu.VMEM_SHARED((NP, HD), jnp.float32),
                       pltpu.VMEM_SHARED((NP, CW), jnp.float32),
                       pltpu.SemaphoreType.DMA],
        mesh=mesh,
        compiler_params=pltpu.CompilerParams(use_tc_tiling_on_sc=False),
    )
    agg, cnt = scatter(msg, tgt, zero_agg, zero_cnt, ones_rows)

    Bn = 2000
    out = pl.pallas_call(
        _combine_body,
        grid=(N // Bn,),
        in_specs=[pl.BlockSpec((NC, Bn, HD), lambda i: (0, i, 0)),
                  pl.BlockSpec((NC, Bn, CW), lambda i: (0, i, 0))],
        out_specs=pl.BlockSpec((Bn, D), lambda i: (i, 0)),
        out_shape=jax.ShapeDtypeStruct((N, D), jnp.float32),
    )(agg, cnt)
    return out


# speedup vs baseline: 4.3347x; 1.6246x over previous
import functools

import jax
import jax.numpy as jnp
from jax import lax
from jax.experimental import pallas as pl
from jax.experimental.pallas import tpu as pltpu
from jax.experimental.pallas import tpu_sc as plsc

N, E, D, FD = 10000, 320000, 128, 64
NC, NS = 2, 16
NW = NC * NS
EPW = E // NW
NP = 10240
NPT = NP // NS
CW = 16
HD = D // 2
EPT = E // NS

C1 = 80
CG = 40
GKG = 5
NGG = EPW // (CG * GKG)
CS = 40
GKS = 5
NGS = EPT // (CS * GKS)


def _gather_body(x_hbm, tgt_hbm, src_hbm, tf_hbm, sf_hbm,
                 idx_all, rows_t, rows_s, gsem, wsem0, wsem1):
    wid = lax.axis_index("s") * NC + lax.axis_index("c")
    base = wid * EPW
    pltpu.sync_copy(tgt_hbm.at[pl.ds(base, EPW)], idx_all.at[0])
    pltpu.sync_copy(src_hbm.at[pl.ds(base, EPW)], idx_all.at[1])

    def phase(g, s):
        wsem = wsem0 if s == 0 else wsem1
        @pl.when(g >= 2)
        def _():
            offp = base + (g - 2) * GKG * CG
            for j in range(GKG):
                pltpu.make_async_copy(
                    rows_t.at[s, j], tf_hbm.at[pl.ds(offp + j * CG, CG)],
                    wsem).wait()
                pltpu.make_async_copy(
                    rows_s.at[s, j], sf_hbm.at[pl.ds(offp + j * CG, CG)],
                    wsem).wait()

        goff = g * GKG * CG
        handles = []
        for j in range(GKG):
            off = goff + j * CG
            handles.append(pltpu.async_copy(
                x_hbm.at[idx_all.at[0, pl.ds(off, CG)]], rows_t.at[s, j], gsem))
            handles.append(pltpu.async_copy(
                x_hbm.at[idx_all.at[1, pl.ds(off, CG)]], rows_s.at[s, j], gsem))
        for h in handles:
            h.wait()
        for j in range(GKG):
            off = base + goff + j * CG
            pltpu.async_copy(rows_t.at[s, j], tf_hbm.at[pl.ds(off, CG)], wsem)
            pltpu.async_copy(rows_s.at[s, j], sf_hbm.at[pl.ds(off, CG)], wsem)

    def body(h, carry):
        phase(2 * h, 0)
        phase(2 * h + 1, 1)
        return carry

    lax.fori_loop(0, NGG // 2, body, 0)
    for g, s in ((NGG - 2, 0), (NGG - 1, 1)):
        wsem = wsem0 if s == 0 else wsem1
        offp = base + g * GKG * CG
        for j in range(GKG):
            pltpu.make_async_copy(
                rows_t.at[s, j], tf_hbm.at[pl.ds(offp + j * CG, CG)],
                wsem).wait()
            pltpu.make_async_copy(
                rows_s.at[s, j], sf_hbm.at[pl.ds(offp + j * CG, CG)],
                wsem).wait()


def _scatter_body(msg_hbm, tgt3d_hbm, zero_agg_hbm, zero_cnt_hbm, ones_hbm,
                  agg_hbm, cnt_hbm,
                  idx2d, rows_v, ones_v, wb_v, wbc_v, agg_sh, cnt_sh,
                  lsem, ssem0, ssem1):
    cid = lax.axis_index("c")
    sid = lax.axis_index("s")
    pltpu.sync_copy(zero_agg_hbm, agg_sh.at[pl.ds(sid * NPT, NPT)])
    pltpu.sync_copy(zero_cnt_hbm, cnt_sh.at[pl.ds(sid * NPT, NPT)])
    pltpu.sync_copy(ones_hbm, ones_v)
    pltpu.sync_copy(tgt3d_hbm.at[sid], idx2d)
    plsc.subcore_barrier()

    base = sid * EPT
    col0 = cid * HD

    def drain_scatters(s):
        ssem = ssem0 if s == 0 else ssem1
        for j in range(GKS):
            pltpu.make_async_copy(
                rows_v.at[s, j], agg_sh.at[pl.ds(0, CS)], ssem).wait()

            @pl.when(cid == 0)
            def _():
                pltpu.make_async_copy(
                    ones_v, cnt_sh.at[pl.ds(0, CS)], ssem).wait()

    def phase(g, s):
        @pl.when(g >= 2)
        def _():
            drain_scatters(s)

        goff = g * GKS * CS
        handles = []
        for j in range(GKS):
            off = base + goff + j * CS
            handles.append(pltpu.async_copy(
                msg_hbm.at[pl.ds(off, CS), pl.ds(col0, HD)],
                rows_v.at[s, j], lsem))
        for h in handles:
            h.wait()
        ssem = ssem0 if s == 0 else ssem1
        for j in range(GKS):
            pltpu.async_copy(
                rows_v.at[s, j], agg_sh.at[idx2d.at[g * GKS + j]], ssem,
                add=True)

            @pl.when(cid == 0)
            def _():
                pltpu.async_copy(
                    ones_v, cnt_sh.at[idx2d.at[g * GKS + j]], ssem,
                    add=True)

    def body(h, carry):
        phase(2 * h, 0)
        phase(2 * h + 1, 1)
        return carry

    lax.fori_loop(0, NGS // 2, body, 0)
    drain_scatters(0)
    drain_scatters(1)
    plsc.subcore_barrier()
    for k in range(4):
        q = NPT // 4
        r0 = sid * NPT + k * q
        pltpu.sync_copy(agg_sh.at[pl.ds(r0, q)], wb_v)
        pltpu.sync_copy(wb_v, agg_hbm.at[cid, pl.ds(r0, q)])
    for k in range(2):
        q = NPT // 2
        r0 = sid * NPT + k * q
        pltpu.sync_copy(cnt_sh.at[pl.ds(r0, q)], wbc_v)
        pltpu.sync_copy(wbc_v, cnt_hbm.at[cid, pl.ds(r0, q)])


def _dense_body(tf_ref, sf_ref, attn_ref, g1_ref, g2_ref, b1_ref, b2_ref,
                w1_ref, w2_ref, bias_ref, out_ref):
    tf = tf_ref[...]
    sf = sf_ref[...]
    s = jnp.sum(tf, axis=1, keepdims=True) + jnp.sum(sf, axis=1, keepdims=True)
    sq = (jnp.sum(tf * tf, axis=1, keepdims=True)
          + jnp.sum(sf * sf, axis=1, keepdims=True))
    mu = s * (1.0 / (2 * D))
    var = sq * (1.0 / (2 * D)) - mu * mu
    inv = lax.rsqrt(var + 1e-5)
    ht = jnp.maximum((tf - mu) * inv * g1_ref[...] + b1_ref[...], 0.0)
    hs = jnp.maximum((sf - mu) * inv * g2_ref[...] + b2_ref[...], 0.0)
    z = (jnp.dot(ht, w1_ref[...], preferred_element_type=jnp.float32)
         + jnp.dot(hs, w2_ref[...], preferred_element_type=jnp.float32)
         + bias_ref[...])
    gate = jnp.mean(jax.nn.sigmoid(z), axis=1, keepdims=True)
    out_ref[...] = sf * (gate * attn_ref[...])


def _combine_body(agg_ref, cnt_ref, out_ref):
    a = jnp.concatenate([agg_ref[0], agg_ref[1]], axis=1)
    c = cnt_ref[0, :, 0:1]
    out_ref[...] = a / jnp.maximum(c, 1.0)


def kernel(x, edge_index, attn_value, ln_gamma, ln_beta, W, b):
    ei = edge_index.astype(jnp.int32)
    tgt = ei[0]
    src = ei[1]

    mesh = plsc.VectorSubcoreMesh(core_axis_name="c", subcore_axis_name="s")
    sc_params = pltpu.CompilerParams(use_tc_tiling_on_sc=False)

    gather = pl.kernel(
        _gather_body,
        out_type=[jax.ShapeDtypeStruct((E, D), jnp.float32),
                  jax.ShapeDtypeStruct((E, D), jnp.float32)],
        scratch_types=[pltpu.VMEM((2, EPW), jnp.int32),
                       pltpu.VMEM((2, GKG, CG, D), jnp.float32),
                       pltpu.VMEM((2, GKG, CG, D), jnp.float32),
                       pltpu.SemaphoreType.DMA,
                       pltpu.SemaphoreType.DMA,
                       pltpu.SemaphoreType.DMA],
        mesh=mesh,
        compiler_params=sc_params,
    )
    tf, sf = gather(x, tgt, src)

    B = 2000
    g1 = ln_gamma[:D].reshape(1, D)
    g2 = ln_gamma[D:].reshape(1, D)
    b1 = ln_beta[:D].reshape(1, D)
    b2 = ln_beta[D:].reshape(1, D)
    w1 = W[:, :D].T
    w2 = W[:, D:].T
    bias = b.reshape(1, FD)
    attn2d = attn_value.reshape(E, 1)

    row_spec = pl.BlockSpec((B, D), lambda i: (i, 0))
    full = lambda shape: pl.BlockSpec(shape, lambda i: tuple(0 for _ in shape))
    msg = pl.pallas_call(
        _dense_body,
        grid=(E // B,),
        in_specs=[row_spec, row_spec,
                  pl.BlockSpec((B, 1), lambda i: (i, 0)),
                  full((1, D)), full((1, D)), full((1, D)), full((1, D)),
                  full((D, FD)), full((D, FD)), full((1, FD))],
        out_specs=row_spec,
        out_shape=jax.ShapeDtypeStruct((E, D), jnp.float32),
    )(tf, sf, attn2d, g1, g2, b1, b2, w1, w2, bias)

    zero_agg = jnp.zeros((NPT, HD), jnp.float32)
    zero_cnt = jnp.zeros((NPT, CW), jnp.float32)
    ones_rows = jnp.ones((CS, CW), jnp.float32)
    tgt3d = tgt.reshape(NS, EPT // CS, CS)
    scatter = pl.kernel(
        _scatter_body,
        out_type=[jax.ShapeDtypeStruct((NC, NP, HD), jnp.float32),
                  jax.ShapeDtypeStruct((NC, NP, CW), jnp.float32)],
        scratch_types=[pltpu.VMEM((EPT // CS, CS), jnp.int32),
                       pltpu.VMEM((2, GKS, CS, HD), jnp.float32),
                       pltpu.VMEM((CS, CW), jnp.float32),
                       pltpu.VMEM((NPT // 4, HD), jnp.float32),
                       pltpu.VMEM((NPT // 2, CW), jnp.float32),
                       pltpu.VMEM_SHARED((NP, HD), jnp.float32),
                       pltpu.VMEM_SHARED((NP, CW), jnp.float32),
                       pltpu.SemaphoreType.DMA,
                       pltpu.SemaphoreType.DMA,
                       pltpu.SemaphoreType.DMA],
        mesh=mesh,
        compiler_params=sc_params,
    )
    agg, cnt = scatter(msg, tgt3d, zero_agg, zero_cnt, ones_rows)

    Bn = 2000
    out = pl.pallas_call(
        _combine_body,
        grid=(N // Bn,),
        in_specs=[pl.BlockSpec((NC, Bn, HD), lambda i: (0, i, 0)),
                  pl.BlockSpec((NC, Bn, CW), lambda i: (0, i, 0))],
        out_specs=pl.BlockSpec((Bn, D), lambda i: (i, 0)),
        out_shape=jax.ShapeDtypeStruct((N, D), jnp.float32),
    )(agg, cnt)
    return out
